# baseline (speedup 1.0000x reference)
import jax
import jax.numpy as jnp
from jax import lax
from jax.experimental import pallas as pl
from jax.experimental.pallas import tpu as pltpu
from jax.experimental.pallas import tpu_sc as plsc

B, T, U, H = 4, 512, 64, 256
NC, NS = 2, 16
NW = NC * NS
ROWS_PER_W = (B * T) // NW
WORKERS_PER_B = T // ROWS_PER_W
L = 16
HC = H // L


def _body(f_hbm, g_hbm, lens_hbm, out_hbm, g_v, f_v, buf_v, z_v, lens_v):
    w = lax.axis_index("s") * NC + lax.axis_index("c")
    b = w // WORKERS_PER_B
    t0 = (w % WORKERS_PER_B) * ROWS_PER_W

    pltpu.sync_copy(lens_hbm.at[w], lens_v)
    pltpu.sync_copy(g_hbm.at[b], g_v)
    pltpu.sync_copy(f_hbm.at[b, pl.ds(t0, ROWS_PER_W)], f_v)

    lv = lens_v[...]
    nt = lv[0]
    glen = lv[1]

    zero = jnp.zeros((L,), jnp.float32)

    def zrow(u, c):
        for j in range(HC):
            buf_v[u, pl.ds(j * L, L)] = zero
            z_v[u, pl.ds(j * L, L)] = zero
        return c

    lax.fori_loop(0, U, zrow, 0)

    def trow(t, c):
        fc = [f_v[t, pl.ds(j * L, L)] for j in range(HC)]

        def urow(u, c2):
            for j in range(HC):
                buf_v[u, pl.ds(j * L, L)] = fc[j] + g_v[u, pl.ds(j * L, L)]
            return c2

        lax.fori_loop(0, glen, urow, 0)
        pltpu.sync_copy(buf_v, out_hbm.at[b, t0 + t])
        return c

    lax.fori_loop(0, nt, trow, 0)

    def ztrow(t, c):
        pltpu.sync_copy(z_v, out_hbm.at[b, t0 + t])
        return c

    lax.fori_loop(nt, ROWS_PER_W, ztrow, 0)


def kernel(f, g, f_len, g_len):
    wids = jnp.arange(NW, dtype=jnp.int32)
    wb = wids // WORKERS_PER_B
    wt0 = (wids % WORKERS_PER_B) * ROWS_PER_W
    nt = jnp.clip(f_len.astype(jnp.int32)[wb] - wt0, 0, ROWS_PER_W)
    gl = g_len.astype(jnp.int32)[wb]
    lens = jnp.zeros((NW, 16), jnp.int32).at[:, 0].set(nt).at[:, 1].set(gl)
    mesh = plsc.VectorSubcoreMesh(
        core_axis_name="c", subcore_axis_name="s", num_cores=NC, num_subcores=NS
    )
    return pl.kernel(
        _body,
        out_type=jax.ShapeDtypeStruct((B, T, U, H), jnp.float32),
        mesh=mesh,
        scratch_types=[
            pltpu.VMEM((U, H), jnp.float32),
            pltpu.VMEM((ROWS_PER_W, H), jnp.float32),
            pltpu.VMEM((U, H), jnp.float32),
            pltpu.VMEM((U, H), jnp.float32),
            pltpu.VMEM((16,), jnp.int32),
        ],
    )(f, g, lens)


# speedup vs baseline: 1.2452x; 1.2452x over previous
import jax
import jax.numpy as jnp
from jax import lax
from jax.experimental import pallas as pl
from jax.experimental.pallas import tpu as pltpu
from jax.experimental.pallas import tpu_sc as plsc

B, T, U, H = 4, 512, 64, 256
NC, NS = 2, 16
NW = NC * NS
ROWS_PER_W = (B * T) // NW
WORKERS_PER_B = T // ROWS_PER_W
L = 16
HC = H // L


def _body(f_hbm, g_hbm, lens_hbm, out_hbm, g_v, f_v, buf0, buf1, lens_v,
          sem0, sem1):
    w = lax.axis_index("s") * NC + lax.axis_index("c")
    b = w // WORKERS_PER_B
    t0 = (w % WORKERS_PER_B) * ROWS_PER_W

    pltpu.sync_copy(lens_hbm.at[w], lens_v)
    pltpu.sync_copy(g_hbm.at[b], g_v)
    pltpu.sync_copy(f_hbm.at[b, pl.ds(t0, ROWS_PER_W)], f_v)

    lv = lens_v[...]
    nt = lv[0]
    glen = lv[1]

    zero = jnp.zeros((L,), jnp.float32)

    def zrow(u, c):
        for j in range(HC):
            buf0[u, pl.ds(j * L, L)] = zero
            buf1[u, pl.ds(j * L, L)] = zero
        return c

    lax.fori_loop(0, U, zrow, 0)

    def fill(bufk, t):
        @pl.when(t < nt)
        def _():
            fc = [f_v[t, pl.ds(j * L, L)] for j in range(HC)]

            def urow(u, c):
                for j in range(HC):
                    bufk[u, pl.ds(j * L, L)] = fc[j] + g_v[u, pl.ds(j * L, L)]
                return c

            lax.fori_loop(0, glen, urow, 0)

        @pl.when(t >= nt)
        def _():
            def uzero(u, c):
                for j in range(HC):
                    bufk[u, pl.ds(j * L, L)] = zero
                return c

            lax.fori_loop(0, glen, uzero, 0)

    def tpair(i, c):
        t = 2 * i

        @pl.when(i >= 1)
        def _():
            pltpu.make_async_copy(buf0, out_hbm.at[b, 0], sem0).wait()

        fill(buf0, t)
        pltpu.async_copy(buf0, out_hbm.at[b, t0 + t], sem0)

        @pl.when(i >= 1)
        def _():
            pltpu.make_async_copy(buf1, out_hbm.at[b, 0], sem1).wait()

        fill(buf1, t + 1)
        pltpu.async_copy(buf1, out_hbm.at[b, t0 + t + 1], sem1)
        return c

    lax.fori_loop(0, ROWS_PER_W // 2, tpair, 0)
    pltpu.make_async_copy(buf0, out_hbm.at[b, 0], sem0).wait()
    pltpu.make_async_copy(buf1, out_hbm.at[b, 0], sem1).wait()


def kernel(f, g, f_len, g_len):
    wids = jnp.arange(NW, dtype=jnp.int32)
    wb = wids // WORKERS_PER_B
    wt0 = (wids % WORKERS_PER_B) * ROWS_PER_W
    nt = jnp.clip(f_len.astype(jnp.int32)[wb] - wt0, 0, ROWS_PER_W)
    gl = g_len.astype(jnp.int32)[wb]
    lens = jnp.zeros((NW, 16), jnp.int32).at[:, 0].set(nt).at[:, 1].set(gl)
    mesh = plsc.VectorSubcoreMesh(
        core_axis_name="c", subcore_axis_name="s", num_cores=NC, num_subcores=NS
    )
    return pl.kernel(
        _body,
        out_type=jax.ShapeDtypeStruct((B, T, U, H), jnp.float32),
        mesh=mesh,
        scratch_types=[
            pltpu.VMEM((U, H), jnp.float32),
            pltpu.VMEM((ROWS_PER_W, H), jnp.float32),
            pltpu.VMEM((U, H), jnp.float32),
            pltpu.VMEM((U, H), jnp.float32),
            pltpu.VMEM((16,), jnp.int32),
            pltpu.SemaphoreType.DMA,
            pltpu.SemaphoreType.DMA,
        ],
    )(f, g, lens)
